# Initial kernel scaffold; baseline (speedup 1.0000x reference)
import jax
import jax.numpy as jnp
from jax.experimental import pallas as pl


def kernel(feat, edge_index, Ws, bs, Wn, bn, Wi, bi, alw, alb, ahw, ahb, amw, amb):
    raise NotImplementedError("write your pallas kernel here")


# speedup vs baseline: 2.7783x; 2.7783x over previous
import functools

import jax
import jax.numpy as jnp
from jax import lax
from jax.experimental import pallas as pl
from jax.experimental.pallas import tpu as pltpu
from jax.experimental.pallas import tpu_sc as plsc

_N = 10000
_E = 160000
_D = 256
_NW = 32
_K = 128
_CH = 40
_CHS = 80
_EPAD = _NW * _CH * _K
_N2 = 10240
_NP = _N2 + 8
_RPT = _N2 // 16


@functools.partial(
    pl.kernel,
    out_type=jax.ShapeDtypeStruct((2, _N2, 128), jnp.float32),
    mesh=plsc.VectorSubcoreMesh(core_axis_name="c", subcore_axis_name="s"),
    scratch_types=[
        pltpu.VMEM_SHARED((_NP, 128), jnp.float32),
        pltpu.VMEM((1, _K), jnp.int32),
        pltpu.VMEM((1, _K), jnp.int32),
        pltpu.VMEM((_K, 128), jnp.float32),
        pltpu.SemaphoreType.DMA,
    ],
)
def _sc_segsum(feat2, srcs01, dsts, summ_out,
               acc, dst_v, idx_v, rows_v, sem):
    c = lax.axis_index("c")
    s = lax.axis_index("s")

    zeros16 = jnp.zeros((16,), jnp.float32)

    def fill_row(r, _):
        def fill_piece(k, _):
            rows_v[r, pl.ds(k * 16, 16)] = zeros16
            return 0
        lax.fori_loop(0, 8, fill_piece, 0)
        return 0
    lax.fori_loop(0, _K, fill_row, 0)

    base = s * _RPT

    def zero_acc(k, _):
        pltpu.sync_copy(rows_v, acc.at[pl.ds(base + k * _K, _K)])
        return 0
    lax.fori_loop(0, _RPT // _K, zero_acc, 0)
    plsc.subcore_barrier()

    def chunk(j, _):
        pltpu.sync_copy(srcs01.at[c, s, pl.ds(j, 1)], idx_v)
        pltpu.sync_copy(dsts.at[s, pl.ds(j, 1)], dst_v)
        pltpu.async_copy(feat2.at[idx_v.at[0]], rows_v, sem).wait()
        pltpu.sync_copy(rows_v, acc.at[dst_v.at[0]], add=True)
        return 0
    lax.fori_loop(0, _CHS, chunk, 0)

    plsc.subcore_barrier()

    def wb(k, _):
        sl = pl.ds(base + k * _K, _K)
        pltpu.sync_copy(acc.at[sl], rows_v)
        pltpu.sync_copy(rows_v, summ_out.at[c, sl])
        return 0
    lax.fori_loop(0, _RPT // _K, wb, 0)


@functools.partial(
    pl.kernel,
    out_type=jax.ShapeDtypeStruct((2, _N2, 128), jnp.float32),
    mesh=plsc.VectorSubcoreMesh(core_axis_name="c", subcore_axis_name="s"),
    scratch_types=[
        pltpu.VMEM_SHARED((_NP, 128), jnp.float32),
        pltpu.VMEM((1, _K), jnp.int32),
        pltpu.VMEM((_K, 128), jnp.float32),
    ],
)
def _sc_count(dsts, cnt_out, accc, dst_v, buf):
    c = lax.axis_index("c")
    s = lax.axis_index("s")
    wid = c * 16 + s

    zeros16 = jnp.zeros((16,), jnp.float32)
    ones16 = jnp.ones((16,), jnp.float32)

    def fill_zero(r, _):
        def piece(k, _):
            buf[r, pl.ds(k * 16, 16)] = zeros16
            return 0
        lax.fori_loop(0, 8, piece, 0)
        return 0
    lax.fori_loop(0, _K, fill_zero, 0)

    base = s * _RPT

    def zero_acc(k, _):
        pltpu.sync_copy(buf, accc.at[pl.ds(base + k * _K, _K)])
        return 0
    lax.fori_loop(0, _RPT // _K, zero_acc, 0)

    def fill_one(r, _):
        def piece(k, _):
            buf[r, pl.ds(k * 16, 16)] = ones16
            return 0
        lax.fori_loop(0, 8, piece, 0)
        return 0
    lax.fori_loop(0, _K, fill_one, 0)
    plsc.subcore_barrier()

    def chunk(j, _):
        pltpu.sync_copy(dsts.at[wid, pl.ds(j, 1)], dst_v)
        pltpu.sync_copy(buf, accc.at[dst_v.at[0]], add=True)
        return 0
    lax.fori_loop(0, _CH, chunk, 0)

    plsc.subcore_barrier()

    def wb(k, _):
        sl = pl.ds(base + k * _K, _K)
        pltpu.sync_copy(accc.at[sl], buf)
        pltpu.sync_copy(buf, cnt_out.at[c, sl])
        return 0
    lax.fori_loop(0, _RPT // _K, wb, 0)


_RB = 1000


def _dense_body(x_ref, s0_ref, s1_ref, c0_ref, c1_ref,
                wst_ref, bs_ref, wnt_ref, bn_ref, wit_ref, bi_ref,
                alw_ref, alb_ref, ahw_ref, ahb_ref, amw_ref, amb_ref,
                o_ref):
    x = x_ref[...]
    cnt = c0_ref[:, 0:1] + c1_ref[:, 0:1]
    inv = 1.0 / jnp.maximum(cnt, 1.0)
    hn = jnp.concatenate([s0_ref[...], s1_ref[...]], axis=1) * inv
    fs = jnp.dot(x, wst_ref[...], preferred_element_type=jnp.float32) + bs_ref[...]
    fn = jnp.dot(hn, wnt_ref[...], preferred_element_type=jnp.float32) + bn_ref[...]
    ident = jnp.maximum(
        jnp.dot(x, wit_ref[...], preferred_element_type=jnp.float32) + bi_ref[...],
        0.0)
    low = jnp.maximum(fs + fn, 0.0)
    high = jnp.maximum(fs - fn, 0.0)

    def att(v, w, b):
        n = jnp.sqrt(jnp.sum(v * v, axis=1, keepdims=True)) + 1e-16
        t = jnp.sum(v * w, axis=1, keepdims=True) / n + b
        e = jnp.where(t > 0.0, t, 5.0 * (jnp.exp(jnp.minimum(t, 0.0)) - 1.0))
        return 1.0 / (1.0 + jnp.exp(-e))

    al = att(low, alw_ref[...], alb_ref[...])
    ah = att(high, ahw_ref[...], ahb_ref[...])
    am = att(ident, amw_ref[...], amb_ref[...])
    o_ref[...] = 3.0 * (al * low + ah * high + am * ident)


def _row_spec(cols):
    return pl.BlockSpec((_RB, cols), lambda i: (i, 0))


def _full_spec(r, cols):
    return pl.BlockSpec((r, cols), lambda i: (0, 0))


_dense = pl.pallas_call(
    _dense_body,
    grid=(_N // _RB,),
    in_specs=[
        _row_spec(_D),
        _row_spec(128),
        _row_spec(128),
        _row_spec(16),
        _row_spec(16),
        _full_spec(_D, _D), _full_spec(1, _D),
        _full_spec(_D, _D), _full_spec(1, _D),
        _full_spec(_D, _D), _full_spec(1, _D),
        _full_spec(1, _D), _full_spec(1, 1),
        _full_spec(1, _D), _full_spec(1, 1),
        _full_spec(1, _D), _full_spec(1, 1),
    ],
    out_specs=_row_spec(_D),
    out_shape=jax.ShapeDtypeStruct((_N, _D), jnp.float32),
)


def kernel(feat, edge_index, Ws, bs, Wn, bn, Wi, bi,
           alw, alb, ahw, ahb, amw, amb):
    src = edge_index[0]
    dst = edge_index[1]
    pad = _EPAD - _E
    src_p = jnp.concatenate([src, jnp.zeros((pad,), jnp.int32)])
    dst_p = jnp.concatenate([dst, jnp.full((pad,), _N2, jnp.int32)])
    src_s = src_p.reshape(16, _CHS, _K)
    dst_s = dst_p.reshape(16, _CHS, _K)
    feat2 = feat.reshape(2 * _N, 128)

    src01 = jnp.stack([2 * src_s, 2 * src_s + 1])
    summ = _sc_segsum(feat2, src01, dst_s)
    cnts = _sc_count(dst_p.reshape(_NW, _CH, _K))

    return _dense(
        feat, summ[0, :_N], summ[1, :_N],
        cnts[0, :_N, :16], cnts[1, :_N, :16],
        Ws.T, bs.reshape(1, _D),
        Wn.T, bn.reshape(1, _D),
        Wi.T, bi.reshape(1, _D),
        alw, alb.reshape(1, 1),
        ahw, ahb.reshape(1, 1),
        amw, amb.reshape(1, 1),
    )


# speedup vs baseline: 3.1826x; 1.1455x over previous
import functools

import jax
import jax.numpy as jnp
from jax import lax
from jax.experimental import pallas as pl
from jax.experimental.pallas import tpu as pltpu
from jax.experimental.pallas import tpu_sc as plsc

_N = 10000
_E = 160000
_D = 256
_NW = 32
_K = 128
_CH = 40
_CHS = 80
_EPAD = _NW * _CH * _K
_N2 = 10240
_NP = _N2 + 8
_RPT = _N2 // 16


_KS = 64
_CHT = _EPAD // (16 * _KS)
_RPT8 = _N2 // 16


@functools.partial(
    pl.kernel,
    out_type=jax.ShapeDtypeStruct((2, _N2, 128), jnp.float32),
    mesh=plsc.VectorSubcoreMesh(core_axis_name="c", subcore_axis_name="s"),
    scratch_types=[
        pltpu.VMEM_SHARED((_NP, 128), jnp.float32),
        pltpu.VMEM((2, _KS), jnp.int32),
        pltpu.VMEM((2, _KS), jnp.int32),
        pltpu.VMEM((2 * _KS, 128), jnp.float32),
        pltpu.SemaphoreType.DMA,
        pltpu.SemaphoreType.DMA,
    ],
)
def _sc_segsum(feat2, srcs01, dsts, summ_out,
               acc, dst_v, idx_v, rows_v, sem0, sem1):
    c = lax.axis_index("c")
    s = lax.axis_index("s")

    zeros16 = jnp.zeros((16,), jnp.float32)

    def fill_row(r, _):
        def fill_piece(k, _):
            rows_v[r, pl.ds(k * 16, 16)] = zeros16
            return 0
        lax.fori_loop(0, 8, fill_piece, 0)
        return 0
    lax.fori_loop(0, _KS, fill_row, 0)

    base = s * _RPT8

    def zero_acc(k, _):
        pltpu.sync_copy(rows_v.at[pl.ds(0, _KS)],
                        acc.at[pl.ds(base + k * _KS, _KS)])
        return 0
    lax.fori_loop(0, _RPT8 // _KS, zero_acc, 0)
    plsc.subcore_barrier()

    sems = (sem0, sem1)

    def start(t, slot):
        pltpu.sync_copy(srcs01.at[c, s, pl.ds(t, 1)],
                        idx_v.at[pl.ds(slot, 1)])
        pltpu.sync_copy(dsts.at[s, pl.ds(t, 1)],
                        dst_v.at[pl.ds(slot, 1)])
        pltpu.async_copy(feat2.at[idx_v.at[slot]],
                         rows_v.at[pl.ds(slot * _KS, _KS)], sems[slot])

    def drain_scatter(slot):
        pltpu.make_async_copy(feat2.at[pl.ds(0, _KS)],
                              rows_v.at[pl.ds(slot * _KS, _KS)],
                              sems[slot]).wait()
        pltpu.sync_copy(rows_v.at[pl.ds(slot * _KS, _KS)],
                        acc.at[dst_v.at[slot]], add=True)

    start(0, 0)

    def pair(t, _):
        start(2 * t + 1, 1)
        drain_scatter(0)

        @pl.when(t + 1 < _CHT // 2)
        def _():
            start(2 * t + 2, 0)

        drain_scatter(1)
        return 0
    lax.fori_loop(0, _CHT // 2, pair, 0)

    plsc.subcore_barrier()

    def wb(k, _):
        sl = pl.ds(base + k * _KS, _KS)
        pltpu.sync_copy(acc.at[sl], rows_v.at[pl.ds(0, _KS)])
        pltpu.sync_copy(rows_v.at[pl.ds(0, _KS)], summ_out.at[c, sl])
        return 0
    lax.fori_loop(0, _RPT8 // _KS, wb, 0)


@functools.partial(
    pl.kernel,
    out_type=jax.ShapeDtypeStruct((2, _N2, 128), jnp.float32),
    mesh=plsc.VectorSubcoreMesh(core_axis_name="c", subcore_axis_name="s"),
    scratch_types=[
        pltpu.VMEM_SHARED((_NP, 128), jnp.float32),
        pltpu.VMEM((1, _K), jnp.int32),
        pltpu.VMEM((_K, 128), jnp.float32),
    ],
)
def _sc_count(dsts, cnt_out, accc, dst_v, buf):
    c = lax.axis_index("c")
    s = lax.axis_index("s")
    wid = c * 16 + s

    zeros16 = jnp.zeros((16,), jnp.float32)
    ones16 = jnp.ones((16,), jnp.float32)

    def fill_zero(r, _):
        def piece(k, _):
            buf[r, pl.ds(k * 16, 16)] = zeros16
            return 0
        lax.fori_loop(0, 8, piece, 0)
        return 0
    lax.fori_loop(0, _K, fill_zero, 0)

    base = s * _RPT

    def zero_acc(k, _):
        pltpu.sync_copy(buf, accc.at[pl.ds(base + k * _K, _K)])
        return 0
    lax.fori_loop(0, _RPT // _K, zero_acc, 0)

    def fill_one(r, _):
        def piece(k, _):
            buf[r, pl.ds(k * 16, 16)] = ones16
            return 0
        lax.fori_loop(0, 8, piece, 0)
        return 0
    lax.fori_loop(0, _K, fill_one, 0)
    plsc.subcore_barrier()

    def chunk(j, _):
        pltpu.sync_copy(dsts.at[wid, pl.ds(j, 1)], dst_v)
        pltpu.sync_copy(buf, accc.at[dst_v.at[0]], add=True)
        return 0
    lax.fori_loop(0, _CH, chunk, 0)

    plsc.subcore_barrier()

    def wb(k, _):
        sl = pl.ds(base + k * _K, _K)
        pltpu.sync_copy(accc.at[sl], buf)
        pltpu.sync_copy(buf, cnt_out.at[c, sl])
        return 0
    lax.fori_loop(0, _RPT // _K, wb, 0)


_RB = 1000


def _dense_body(x_ref, s0_ref, s1_ref, c0_ref, c1_ref,
                wst_ref, bs_ref, wnt_ref, bn_ref, wit_ref, bi_ref,
                alw_ref, alb_ref, ahw_ref, ahb_ref, amw_ref, amb_ref,
                o_ref):
    x = x_ref[...]
    cnt = c0_ref[:, 0:1] + c1_ref[:, 0:1]
    inv = 1.0 / jnp.maximum(cnt, 1.0)
    hn = jnp.concatenate([s0_ref[...], s1_ref[...]], axis=1) * inv
    fs = jnp.dot(x, wst_ref[...], preferred_element_type=jnp.float32) + bs_ref[...]
    fn = jnp.dot(hn, wnt_ref[...], preferred_element_type=jnp.float32) + bn_ref[...]
    ident = jnp.maximum(
        jnp.dot(x, wit_ref[...], preferred_element_type=jnp.float32) + bi_ref[...],
        0.0)
    low = jnp.maximum(fs + fn, 0.0)
    high = jnp.maximum(fs - fn, 0.0)

    def att(v, w, b):
        n = jnp.sqrt(jnp.sum(v * v, axis=1, keepdims=True)) + 1e-16
        t = jnp.sum(v * w, axis=1, keepdims=True) / n + b
        e = jnp.where(t > 0.0, t, 5.0 * (jnp.exp(jnp.minimum(t, 0.0)) - 1.0))
        return 1.0 / (1.0 + jnp.exp(-e))

    al = att(low, alw_ref[...], alb_ref[...])
    ah = att(high, ahw_ref[...], ahb_ref[...])
    am = att(ident, amw_ref[...], amb_ref[...])
    o_ref[...] = 3.0 * (al * low + ah * high + am * ident)


def _row_spec(cols):
    return pl.BlockSpec((_RB, cols), lambda i: (i, 0))


def _full_spec(r, cols):
    return pl.BlockSpec((r, cols), lambda i: (0, 0))


_dense = pl.pallas_call(
    _dense_body,
    grid=(_N // _RB,),
    in_specs=[
        _row_spec(_D),
        _row_spec(128),
        _row_spec(128),
        _row_spec(16),
        _row_spec(16),
        _full_spec(_D, _D), _full_spec(1, _D),
        _full_spec(_D, _D), _full_spec(1, _D),
        _full_spec(_D, _D), _full_spec(1, _D),
        _full_spec(1, _D), _full_spec(1, 1),
        _full_spec(1, _D), _full_spec(1, 1),
        _full_spec(1, _D), _full_spec(1, 1),
    ],
    out_specs=_row_spec(_D),
    out_shape=jax.ShapeDtypeStruct((_N, _D), jnp.float32),
)


def kernel(feat, edge_index, Ws, bs, Wn, bn, Wi, bi,
           alw, alb, ahw, ahb, amw, amb):
    src = edge_index[0]
    dst = edge_index[1]
    pad = _EPAD - _E
    src_p = jnp.concatenate([src, jnp.zeros((pad,), jnp.int32)])
    dst_p = jnp.concatenate([dst, jnp.full((pad,), _N2, jnp.int32)])
    src_s = src_p.reshape(16, _CHT, _KS)
    dst_s = dst_p.reshape(16, _CHT, _KS)
    feat2 = feat.reshape(2 * _N, 128)

    src01 = jnp.stack([2 * src_s, 2 * src_s + 1])
    summ = _sc_segsum(feat2, src01, dst_s)
    cnts = _sc_count(dst_p.reshape(_NW, _CH, _K))

    return _dense(
        feat, summ[0, :_N], summ[1, :_N],
        cnts[0, :_N, :16], cnts[1, :_N, :16],
        Ws.T, bs.reshape(1, _D),
        Wn.T, bn.reshape(1, _D),
        Wi.T, bi.reshape(1, _D),
        alw, alb.reshape(1, 1),
        ahw, ahb.reshape(1, 1),
        amw, amb.reshape(1, 1),
    )
